# Initial kernel scaffold; baseline (speedup 1.0000x reference)
import jax
import jax.numpy as jnp
from jax.experimental import pallas as pl


def kernel(input_ids, embed_tokens):
    raise NotImplementedError("write your pallas kernel here")


# speedup vs baseline: 1.7658x; 1.7658x over previous
import functools

import jax
import jax.numpy as jnp
from jax import lax
from jax.experimental import pallas as pl
from jax.experimental.pallas import tpu as pltpu
from jax.experimental.pallas import tpu_sc as plsc

_BATCH = 4
_SEQ = 8192
_D = 1024
_N = _BATCH * _SEQ


def _sc_geometry():
    try:
        info = plsc.get_sparse_core_info()
        return info.num_cores, info.num_subcores
    except Exception:
        return 2, 16


@functools.lru_cache(maxsize=None)
def _build(vocab: int, d: int, n: int):
    nc, ns = _sc_geometry()
    nw = nc * ns
    n_per_w = n // nw
    ch = 32
    nch = n_per_w // ch
    assert n_per_w % ch == 0

    mesh = plsc.VectorSubcoreMesh(core_axis_name="c", subcore_axis_name="s")

    @functools.partial(
        pl.kernel,
        mesh=mesh,
        out_type=jax.ShapeDtypeStruct((n, d), jnp.float32),
        scratch_types=[
            pltpu.VMEM((nch, ch), jnp.int32),
            pltpu.VMEM((ch, d), jnp.float32),
            pltpu.VMEM((ch, d), jnp.float32),
            pltpu.SemaphoreType.DMA,
            pltpu.SemaphoreType.DMA,
        ],
    )
    def gather_kernel(table_hbm, idx_hbm, out_hbm, idx_v, buf0, buf1, sem0, sem1):
        wid = lax.axis_index("s") * nc + lax.axis_index("c")
        base = wid * n_per_w
        bufs = (buf0, buf1)
        sems = (sem0, sem1)

        pltpu.sync_copy(idx_hbm.at[wid], idx_v)

        pltpu.async_copy(table_hbm.at[idx_v.at[0]], buf0, sem0)

        def step(g, _):
            for b in range(2):
                c = g + b
                nb = (b + 1) % 2

                @pl.when(c + 1 < nch)
                def _():
                    pltpu.async_copy(
                        table_hbm.at[idx_v.at[c + 1]], bufs[nb], sems[nb]
                    )

                pltpu.make_async_copy(
                    table_hbm.at[idx_v.at[c]], bufs[b], sems[b]
                ).wait()
                pltpu.sync_copy(bufs[b], out_hbm.at[pl.ds(base + c * ch, ch)])
            return ()

        lax.fori_loop(0, nch // 2, lambda i, c: step(i * 2, c), (), unroll=False)

    return gather_kernel


def kernel(input_ids, embed_tokens):
    vocab, d = embed_tokens.shape
    nc, ns = _sc_geometry()
    nw = nc * ns
    ids = input_ids.reshape(-1).astype(jnp.int32)
    n = ids.shape[0]
    ch = 32
    idx = ids.reshape(nw, (n // nw) // ch, ch)
    out = _build(vocab, d, n)(embed_tokens, idx)
    return out.reshape(*input_ids.shape, d)
